# baseline (speedup 1.0000x reference)
import functools

import jax
import jax.numpy as jnp
from jax import lax
from jax.experimental import pallas as pl
from jax.experimental.pallas import tpu as pltpu
from jax.experimental.pallas import tpu_sc as plsc

F = 26
V = 100000
D = 32
B = 16384
N = F * V
CH = 2080


def _depad_body(x_ref, o_ref):
    o_ref[:, :D] = x_ref[...]


def _widen_tables(tables2d):
    return pl.pallas_call(
        _depad_body,
        grid=(N // CH,),
        in_specs=[pl.BlockSpec((CH, D), lambda i: (i, 0))],
        out_specs=pl.BlockSpec((CH, 128), lambda i: (i, 0)),
        out_shape=jax.ShapeDtypeStruct((N, 128), jnp.float32),
    )(tables2d)


def _make_emb_kernel():
    info = plsc.get_sparse_core_info()
    NC, NS = info.num_cores, info.num_subcores
    NW = NC * NS
    BPW = B // NW
    HC = 256

    mesh = plsc.VectorSubcoreMesh(core_axis_name="c", subcore_axis_name="s")

    @functools.partial(
        pl.kernel,
        mesh=mesh,
        out_type=jax.ShapeDtypeStruct((F * B, 128), jnp.float32),
        scratch_types=[
            pltpu.VMEM((BPW, F), jnp.int32),
            pltpu.VMEM((F, BPW), jnp.int32),
            pltpu.VMEM((2, HC, 128), jnp.float32),
            pltpu.SemaphoreType.DMA,
            pltpu.SemaphoreType.DMA,
            pltpu.SemaphoreType.DMA,
            pltpu.SemaphoreType.DMA,
        ],
        compiler_params=pltpu.CompilerParams(
            use_tc_tiling_on_sc=False, needs_layout_passes=False
        ),
    )
    def emb(fv_hbm, tab_hbm, out_hbm, fv_v, idx_v, rows_v, sg0, sg1, sw0, sw1):
        wid = lax.axis_index("s") * NC + lax.axis_index("c")
        base = wid * BPW
        pltpu.sync_copy(fv_hbm.at[pl.ds(base, BPW)], fv_v)

        def tr_body(j, carry):
            rows = lax.iota(jnp.int32, 16) + j * 16
            for i in range(F):
                col = jnp.full((16,), i, jnp.int32)
                v = plsc.load_gather(fv_v, [rows, col]) + (i * V)
                idx_v[i, pl.ds(pl.multiple_of(j * 16, 16), 16)] = v
            return carry

        lax.fori_loop(0, BPW // 16, tr_body, 0)
        sg = (sg0, sg1)
        sw = (sw0, sw1)
        wb = [None, None]
        step = 0
        for i in range(F):
            for h in range(BPW // HC):
                b = step % 2
                if wb[b] is not None:
                    wb[b].wait()
                pltpu.async_copy(
                    tab_hbm.at[idx_v.at[i, pl.ds(h * HC, HC)]],
                    rows_v.at[b],
                    sg[b],
                ).wait()
                wb[b] = pltpu.async_copy(
                    rows_v.at[b],
                    out_hbm.at[pl.ds(i * B + base + h * HC, HC)],
                    sw[b],
                )
                step += 1
        wb[0].wait()
        wb[1].wait()

    return emb


def kernel(feature_value, tables):
    tab_wide = _widen_tables(tables.reshape(N, D))
    s = _make_emb_kernel()(feature_value, tab_wide)
    return s.reshape(F, B, 128)[:, :, :D]


# speedup vs baseline: 1.3878x; 1.3878x over previous
import functools

import jax
import jax.numpy as jnp
from jax import lax
from jax.experimental import pallas as pl
from jax.experimental.pallas import tpu as pltpu
from jax.experimental.pallas import tpu_sc as plsc


def _make_emb_kernel(F, V, D, B):
    info = plsc.get_sparse_core_info()
    NC, NS = info.num_cores, info.num_subcores
    NW = NC * NS
    assert B % NW == 0
    BPW = B // NW

    mesh = plsc.VectorSubcoreMesh(core_axis_name="c", subcore_axis_name="s")

    @functools.partial(
        pl.kernel,
        mesh=mesh,
        out_type=jax.ShapeDtypeStruct((F * B, 128), jnp.float32),
        scratch_types=[
            pltpu.VMEM((BPW, F), jnp.int32),
            pltpu.VMEM((F, BPW), jnp.int32),
            pltpu.VMEM((2, BPW, D), jnp.float32),
            pltpu.SemaphoreType.DMA,
            pltpu.SemaphoreType.DMA,
            pltpu.SemaphoreType.DMA,
            pltpu.SemaphoreType.DMA,
        ],
        compiler_params=pltpu.CompilerParams(
            use_tc_tiling_on_sc=False, needs_layout_passes=False
        ),
    )
    def emb(fv_hbm, tab_hbm, out_hbm, fv_v, idx_v, rows_v, sg0, sg1, sw0, sw1):
        wid = lax.axis_index("s") * NC + lax.axis_index("c")
        base = wid * BPW
        pltpu.sync_copy(fv_hbm.at[pl.ds(base, BPW)], fv_v)

        def tr_body(j, carry):
            rows = lax.iota(jnp.int32, 16) + j * 16
            for i in range(F):
                col = jnp.full((16,), i, jnp.int32)
                v = plsc.load_gather(fv_v, [rows, col])
                idx_v[i, pl.ds(pl.multiple_of(j * 16, 16), 16)] = v
            return carry

        lax.fori_loop(0, BPW // 16, tr_body, 0)
        sg = (sg0, sg1)
        sw = (sw0, sw1)
        wb = [None, None]
        for i in range(F):
            b = i % 2
            if wb[b] is not None:
                wb[b].wait()
            pltpu.async_copy(tab_hbm.at[i].at[idx_v.at[i]], rows_v.at[b], sg[b]).wait()
            wb[b] = pltpu.async_copy(
                rows_v.at[b],
                out_hbm.at[pl.ds(i * B + base, BPW), pl.ds(0, D)],
                sw[b],
            )
        wb[0].wait()
        wb[1].wait()

    return emb


def kernel(feature_value, tables):
    F, V, D = tables.shape
    B = feature_value.shape[0]
    emb = _make_emb_kernel(F, V, D, B)
    s = emb(feature_value, tables)
    return s.reshape(F, B, 128)[:, :, :D]
